# Initial kernel scaffold; baseline (speedup 1.0000x reference)
import jax
import jax.numpy as jnp
from jax.experimental import pallas as pl


def kernel(token_ids, word_table, pos_emb, gamma, beta):
    raise NotImplementedError("write your pallas kernel here")


# speedup vs baseline: 1.5464x; 1.5464x over previous
import functools

import jax
import jax.numpy as jnp
from jax import lax
from jax.experimental import pallas as pl
from jax.experimental.pallas import tpu as pltpu
from jax.experimental.pallas import tpu_sc as plsc

VOCAB = 100000
D_MODEL = 768
MAX_LEN = 2048
BATCH = 4

_NC = 2
_NS = 16
_NW = _NC * _NS

_CHUNK = 64


def _sc_gather(table, flat_ids):
    b = flat_ids.shape[0]
    b_per_w = b // _NW
    n_chunks = b_per_w // _CHUNK
    mesh = plsc.VectorSubcoreMesh(core_axis_name="c", subcore_axis_name="s")

    @functools.partial(
        pl.kernel,
        mesh=mesh,
        out_type=jax.ShapeDtypeStruct((b, D_MODEL), jnp.float32),
        scratch_types=[
            pltpu.VMEM((_CHUNK,), jnp.int32),
            pltpu.VMEM((_CHUNK, D_MODEL), jnp.float32),
            pltpu.SemaphoreType.DMA,
        ],
    )
    def gather_kernel(table_hbm, idx_hbm, out_hbm, idx_v, rows_v, sem):
        wid = lax.axis_index("s") * _NC + lax.axis_index("c")
        base = wid * b_per_w

        @pl.loop(0, n_chunks)
        def _(c):
            off = base + c * _CHUNK
            pltpu.sync_copy(idx_hbm.at[pl.ds(off, _CHUNK)], idx_v)
            pltpu.async_copy(table_hbm.at[idx_v], rows_v, sem).wait()
            pltpu.sync_copy(rows_v, out_hbm.at[pl.ds(off, _CHUNK)])

    return gather_kernel(table, flat_ids)


_LN_BLK = 512


def _ln_body(x_ref, pos_ref, gamma_ref, beta_ref, o_ref):
    x = x_ref[...] + pos_ref[...]
    mean = jnp.mean(x, axis=-1, keepdims=True)
    xc = x - mean
    var = jnp.mean(xc * xc, axis=-1, keepdims=True)
    o_ref[...] = xc * lax.rsqrt(var + 1e-5) * gamma_ref[...] + beta_ref[...]


def _tc_add_ln(gathered, pos_emb, gamma, beta):
    b = gathered.shape[0]
    grid = b // _LN_BLK
    blocks_per_seq = MAX_LEN // _LN_BLK
    return pl.pallas_call(
        _ln_body,
        grid=(grid,),
        in_specs=[
            pl.BlockSpec((_LN_BLK, D_MODEL), lambda i: (i, 0)),
            pl.BlockSpec((_LN_BLK, D_MODEL),
                         lambda i: (i % blocks_per_seq, 0)),
            pl.BlockSpec((1, D_MODEL), lambda i: (0, 0)),
            pl.BlockSpec((1, D_MODEL), lambda i: (0, 0)),
        ],
        out_specs=pl.BlockSpec((_LN_BLK, D_MODEL), lambda i: (i, 0)),
        out_shape=jax.ShapeDtypeStruct((b, D_MODEL), jnp.float32),
    )(gathered, pos_emb, gamma, beta)


@jax.jit
def kernel(token_ids, word_table, pos_emb, gamma, beta):
    flat_ids = token_ids.reshape(-1).astype(jnp.int32)
    gathered = _sc_gather(word_table, flat_ids)
    out = _tc_add_ln(gathered, pos_emb,
                     gamma.reshape(1, D_MODEL), beta.reshape(1, D_MODEL))
    return out.reshape(token_ids.shape[0], -1, D_MODEL)


# speedup vs baseline: 1.5955x; 1.0318x over previous
import functools

import jax
import jax.numpy as jnp
from jax import lax
from jax.experimental import pallas as pl
from jax.experimental.pallas import tpu as pltpu
from jax.experimental.pallas import tpu_sc as plsc

VOCAB = 100000
D_MODEL = 768
MAX_LEN = 2048
BATCH = 4

_NC = 2
_NS = 16
_NW = _NC * _NS

_CHUNK = 64


def _sc_gather(table, flat_ids):
    b = flat_ids.shape[0]
    b_per_w = b // _NW
    n_chunks = b_per_w // _CHUNK
    mesh = plsc.VectorSubcoreMesh(core_axis_name="c", subcore_axis_name="s")

    @functools.partial(
        pl.kernel,
        mesh=mesh,
        out_type=jax.ShapeDtypeStruct((b, D_MODEL), jnp.float32),
        scratch_types=[
            pltpu.VMEM((b_per_w,), jnp.int32),
            pltpu.VMEM((_CHUNK, D_MODEL), jnp.float32),
            pltpu.VMEM((_CHUNK, D_MODEL), jnp.float32),
            pltpu.SemaphoreType.DMA,
            pltpu.SemaphoreType.DMA,
            pltpu.SemaphoreType.DMA,
            pltpu.SemaphoreType.DMA,
        ],
    )
    def gather_kernel(table_hbm, idx_hbm, out_hbm, idx_v,
                      rows0, rows1, g0, g1, w0, w1):
        wid = lax.axis_index("s") * _NC + lax.axis_index("c")
        base = wid * b_per_w
        pltpu.sync_copy(idx_hbm.at[pl.ds(base, b_per_w)], idx_v)

        bufs = (rows0, rows1)
        gsems = (g0, g1)
        wsems = (w0, w1)
        g_copies = [None] * n_chunks
        w_copies = [None] * n_chunks

        def start_gather(c):
            g_copies[c] = pltpu.async_copy(
                table_hbm.at[idx_v.at[pl.ds(c * _CHUNK, _CHUNK)]],
                bufs[c % 2], gsems[c % 2])

        def start_write(c):
            w_copies[c] = pltpu.async_copy(
                bufs[c % 2], out_hbm.at[pl.ds(base + c * _CHUNK, _CHUNK)],
                wsems[c % 2])

        start_gather(0)
        if n_chunks > 1:
            start_gather(1)
        for c in range(n_chunks):
            g_copies[c].wait()
            start_write(c)
            nxt = c + 2
            if nxt < n_chunks:
                w_copies[c].wait()
                start_gather(nxt)
        for c in range(max(0, n_chunks - 2), n_chunks):
            w_copies[c].wait()

    return gather_kernel(table, flat_ids)


_LN_BLK = 512


def _ln_body(x_ref, pos_ref, gamma_ref, beta_ref, o_ref):
    x = x_ref[...] + pos_ref[...]
    mean = jnp.mean(x, axis=-1, keepdims=True)
    xc = x - mean
    var = jnp.mean(xc * xc, axis=-1, keepdims=True)
    o_ref[...] = xc * lax.rsqrt(var + 1e-5) * gamma_ref[...] + beta_ref[...]


def _tc_add_ln(gathered, pos_emb, gamma, beta):
    b = gathered.shape[0]
    grid = b // _LN_BLK
    blocks_per_seq = MAX_LEN // _LN_BLK
    return pl.pallas_call(
        _ln_body,
        grid=(grid,),
        in_specs=[
            pl.BlockSpec((_LN_BLK, D_MODEL), lambda i: (i, 0)),
            pl.BlockSpec((_LN_BLK, D_MODEL),
                         lambda i: (i % blocks_per_seq, 0)),
            pl.BlockSpec((1, D_MODEL), lambda i: (0, 0)),
            pl.BlockSpec((1, D_MODEL), lambda i: (0, 0)),
        ],
        out_specs=pl.BlockSpec((_LN_BLK, D_MODEL), lambda i: (i, 0)),
        out_shape=jax.ShapeDtypeStruct((b, D_MODEL), jnp.float32),
    )(gathered, pos_emb, gamma, beta)


@jax.jit
def kernel(token_ids, word_table, pos_emb, gamma, beta):
    flat_ids = token_ids.reshape(-1).astype(jnp.int32)
    gathered = _sc_gather(word_table, flat_ids)
    out = _tc_add_ln(gathered, pos_emb,
                     gamma.reshape(1, D_MODEL), beta.reshape(1, D_MODEL))
    return out.reshape(token_ids.shape[0], -1, D_MODEL)


# speedup vs baseline: 1.7737x; 1.1117x over previous
import functools

import jax
import jax.numpy as jnp
from jax import lax
from jax.experimental import pallas as pl
from jax.experimental.pallas import tpu as pltpu
from jax.experimental.pallas import tpu_sc as plsc

VOCAB = 100000
D_MODEL = 768
MAX_LEN = 2048
BATCH = 4

_NC = 2
_NS = 16
_NW = _NC * _NS

_CHUNK = 64


def _sc_gather(table, flat_ids):
    b = flat_ids.shape[0]
    b_per_w = b // _NW
    n_chunks = b_per_w // _CHUNK
    mesh = plsc.VectorSubcoreMesh(core_axis_name="c", subcore_axis_name="s")

    @functools.partial(
        pl.kernel,
        mesh=mesh,
        out_type=jax.ShapeDtypeStruct((b, D_MODEL), jnp.float32),
        scratch_types=[
            pltpu.VMEM((b_per_w,), jnp.int32),
            pltpu.VMEM((_CHUNK, D_MODEL), jnp.float32),
            pltpu.VMEM((_CHUNK, D_MODEL), jnp.float32),
            pltpu.SemaphoreType.DMA,
            pltpu.SemaphoreType.DMA,
            pltpu.SemaphoreType.DMA,
            pltpu.SemaphoreType.DMA,
        ],
    )
    def gather_kernel(table_hbm, idx_hbm, out_hbm, idx_v,
                      rows0, rows1, g0, g1, w0, w1):
        wid = lax.axis_index("s") * _NC + lax.axis_index("c")
        base = wid * b_per_w
        pltpu.sync_copy(idx_hbm.at[pl.ds(base, b_per_w)], idx_v)

        bufs = (rows0, rows1)
        gsems = (g0, g1)
        wsems = (w0, w1)
        g_copies = [None] * n_chunks
        w_copies = [None] * n_chunks

        def start_gather(c):
            g_copies[c] = pltpu.async_copy(
                table_hbm.at[idx_v.at[pl.ds(c * _CHUNK, _CHUNK)]],
                bufs[c % 2], gsems[c % 2])

        def start_write(c):
            w_copies[c] = pltpu.async_copy(
                bufs[c % 2], out_hbm.at[pl.ds(base + c * _CHUNK, _CHUNK)],
                wsems[c % 2])

        start_gather(0)
        if n_chunks > 1:
            start_gather(1)
        for c in range(n_chunks):
            g_copies[c].wait()
            start_write(c)
            nxt = c + 2
            if nxt < n_chunks:
                w_copies[c].wait()
                start_gather(nxt)
        for c in range(max(0, n_chunks - 2), n_chunks):
            w_copies[c].wait()

    return gather_kernel(table, flat_ids)


def _ln_body(x_ref, pos_ref, gamma_ref, beta_ref, o_ref):
    x = x_ref[...] + pos_ref[...]
    mean = jnp.mean(x, axis=-1, keepdims=True)
    xc = x - mean
    var = jnp.mean(xc * xc, axis=-1, keepdims=True)
    o_ref[0] = xc * lax.rsqrt(var + 1e-5) * gamma_ref[...] + beta_ref[...]


def _tc_add_ln(gathered, pos_emb, gamma, beta, n_batch):
    return pl.pallas_call(
        _ln_body,
        grid=(n_batch,),
        in_specs=[
            pl.BlockSpec((MAX_LEN, D_MODEL), lambda b: (b, 0)),
            pl.BlockSpec((MAX_LEN, D_MODEL), lambda b: (0, 0)),
            pl.BlockSpec((1, D_MODEL), lambda b: (0, 0)),
            pl.BlockSpec((1, D_MODEL), lambda b: (0, 0)),
        ],
        out_specs=pl.BlockSpec((1, MAX_LEN, D_MODEL), lambda b: (b, 0, 0)),
        out_shape=jax.ShapeDtypeStruct((n_batch, MAX_LEN, D_MODEL),
                                       jnp.float32),
    )(gathered, pos_emb, gamma, beta)


@jax.jit
def kernel(token_ids, word_table, pos_emb, gamma, beta):
    flat_ids = token_ids.reshape(-1).astype(jnp.int32)
    gathered = _sc_gather(word_table, flat_ids)
    return _tc_add_ln(gathered, pos_emb,
                      gamma.reshape(1, D_MODEL), beta.reshape(1, D_MODEL),
                      token_ids.shape[0])
